# baseline (speedup 1.0000x reference)
import functools
import numpy as np
import jax
import jax.numpy as jnp
from jax import lax
from jax.experimental import pallas as pl
from jax.experimental.pallas import tpu as pltpu
from jax.experimental.pallas import tpu_sc as plsc

_D = 256
_M = 8
_L = 4
_P = 4
_DH = 32
_DF = 1024
_VID = (8192, 4096, 2048, 1024)
_AUD = (4096, 2048, 1024, 512)
_NW = 32
_CH = 4
_BQ = 512


def _value_body(src_ref, maskf_ref, w_ref, b_ref, out_ref):
    x = src_ref[0]
    v = jnp.dot(x, w_ref[...], preferred_element_type=jnp.float32) + b_ref[...]
    out_ref[0] = v * maskf_ref[0]


def _value_proj(src, maskf, w_t, b):
    n, lin, _ = src.shape
    return pl.pallas_call(
        _value_body,
        grid=(n, lin // _BQ),
        in_specs=[
            pl.BlockSpec((1, _BQ, _D), lambda i, j: (i, j, 0)),
            pl.BlockSpec((1, _BQ, 1), lambda i, j: (i, j, 0)),
            pl.BlockSpec((_D, _D), lambda i, j: (0, 0)),
            pl.BlockSpec((1, _D), lambda i, j: (0, 0)),
        ],
        out_specs=pl.BlockSpec((1, _BQ, _D), lambda i, j: (i, j, 0)),
        out_shape=jax.ShapeDtypeStruct((n, lin, _D), jnp.float32),
    )(src, maskf, w_t, b)


def _samp_body(q_ref, refe_ref, w_ref, b_ref, tcol_ref, ibase_ref,
               loc_ref, aw_ref, idx_ref, w0_ref, w1o_ref, w2_ref, *, lin):
    n = pl.program_id(0)
    q = q_ref[0]
    so_aw = jnp.dot(q, w_ref[...], preferred_element_type=jnp.float32) + b_ref[...]
    so = so_aw[:, :128]
    awl = so_aw[:, 128:]
    ri = lax.broadcasted_iota(jnp.int32, (128, 128), 0) // 16
    ci = lax.broadcasted_iota(jnp.int32, (128, 128), 1) // 16
    seg = (ri == ci).astype(jnp.float32)
    e = jnp.exp(awl)
    aw = e / jnp.dot(e, seg, preferred_element_type=jnp.float32)
    tcol = tcol_ref[...]
    loc = refe_ref[0] + so / tcol
    x = loc * tcol - 0.5
    x0f = jnp.floor(x)
    w1 = x - x0f
    t0 = (x0f >= 0.0) & (x0f <= tcol - 1.0)
    t1 = (x0f >= -1.0) & (x0f <= tcol - 2.0)
    wa = aw * jnp.where(t0, 1.0 - w1, jnp.where(t1, w1, 0.0))
    wb = aw * jnp.where(t0 & t1, w1, 0.0)
    r = jnp.clip(x0f, 0.0, tcol - 1.0).astype(jnp.int32)
    gidx = r + ibase_ref[...] + n * (_M * lin)
    podd = (gidx & 1) == 1
    zero = jnp.zeros_like(wa)
    loc_ref[0] = loc
    aw_ref[0] = aw
    idx_ref[0] = gidx >> 1
    w0_ref[0] = jnp.where(podd, zero, wa)
    w1o_ref[0] = jnp.where(podd, wa, wb)
    w2_ref[0] = jnp.where(podd, wb, zero)


def _samp(query, refe, cat_w, cat_b, tcol, ibase, lin):
    n, lq, _ = query.shape
    grid = (n, lq // _BQ)
    blk = pl.BlockSpec((1, _BQ, 128), lambda i, j: (i, j, 0))
    out_shapes = [jax.ShapeDtypeStruct((n, lq, 128), jnp.float32)] * 2 + \
                 [jax.ShapeDtypeStruct((n, lq, 128), jnp.int32)] + \
                 [jax.ShapeDtypeStruct((n, lq, 128), jnp.float32)] * 3
    return pl.pallas_call(
        functools.partial(_samp_body, lin=lin),
        grid=grid,
        in_specs=[
            pl.BlockSpec((1, _BQ, _D), lambda i, j: (i, j, 0)),
            pl.BlockSpec((1, _BQ, 128), lambda i, j: (i, j, 0)),
            pl.BlockSpec((_D, _D), lambda i, j: (0, 0)),
            pl.BlockSpec((1, _D), lambda i, j: (0, 0)),
            pl.BlockSpec((1, 128), lambda i, j: (0, 0)),
            pl.BlockSpec((1, 128), lambda i, j: (0, 0)),
        ],
        out_specs=[blk] * 6,
        out_shape=out_shapes,
    )(query, refe, cat_w, cat_b, tcol, ibase)


def _outln_body(acc_ref, src_ref, w_ref, b_ref, g_ref, bb_ref, o_ref):
    a = acc_ref[0]
    y = jnp.dot(a, w_ref[...], preferred_element_type=jnp.float32) + b_ref[...]
    x = src_ref[0] + y
    mu = jnp.mean(x, -1, keepdims=True)
    var = jnp.mean((x - mu) ** 2, -1, keepdims=True)
    o_ref[0] = (x - mu) / jnp.sqrt(var + 1e-5) * g_ref[...] + bb_ref[...]


def _outln(acc, src, w_t, b, g, bb):
    n, lq, _ = acc.shape
    blk = pl.BlockSpec((1, _BQ, _D), lambda i, j: (i, j, 0))
    vec = pl.BlockSpec((1, _D), lambda i, j: (0, 0))
    return pl.pallas_call(
        _outln_body,
        grid=(n, lq // _BQ),
        in_specs=[blk, blk, pl.BlockSpec((_D, _D), lambda i, j: (0, 0)),
                  vec, vec, vec],
        out_specs=blk,
        out_shape=jax.ShapeDtypeStruct((n, lq, _D), jnp.float32),
    )(acc, src, w_t, b, g, bb)


def _outffn_body(acc_ref, ow_ref, ob_ref, w1_ref, b1_ref, w2_ref, b2_ref,
                 g_ref, bb_ref, o_ref):
    a = acc_ref[0]
    x = jnp.dot(a, ow_ref[...], preferred_element_type=jnp.float32) + ob_ref[...]
    h = jnp.maximum(
        jnp.dot(x, w1_ref[...], preferred_element_type=jnp.float32) + b1_ref[...],
        0.0)
    y = jnp.dot(h, w2_ref[...], preferred_element_type=jnp.float32) + b2_ref[...]
    x = x + y
    mu = jnp.mean(x, -1, keepdims=True)
    var = jnp.mean((x - mu) ** 2, -1, keepdims=True)
    o_ref[0] = (x - mu) / jnp.sqrt(var + 1e-5) * g_ref[...] + bb_ref[...]


def _outffn(acc, ow_t, ob, w1_t, b1, w2_t, b2, g, bb):
    n, lq, _ = acc.shape
    blk = pl.BlockSpec((1, _BQ, _D), lambda i, j: (i, j, 0))
    vec = pl.BlockSpec((1, _D), lambda i, j: (0, 0))
    return pl.pallas_call(
        _outffn_body,
        grid=(n, lq // _BQ),
        in_specs=[blk,
                  pl.BlockSpec((_D, _D), lambda i, j: (0, 0)), vec,
                  pl.BlockSpec((_D, _DF), lambda i, j: (0, 0)),
                  pl.BlockSpec((1, _DF), lambda i, j: (0, 0)),
                  pl.BlockSpec((_DF, _D), lambda i, j: (0, 0)), vec,
                  vec, vec],
        out_specs=blk,
        out_shape=jax.ShapeDtypeStruct((n, lq, _D), jnp.float32),
    )(acc, ow_t, ob, w1_t, b1, w2_t, b2, g, bb)


_GTR_DNUMS = lax.GatherDimensionNumbers(
    offset_dims=(), collapsed_slice_dims=(0,), start_index_map=(0,))


def _bcast(vec, j):
    idx = jnp.full((16, 1), j, jnp.int32)
    return lax.gather(vec, idx, _GTR_DNUMS, slice_sizes=(1,),
                      mode=lax.GatherScatterMode.PROMISE_IN_BOUNDS)


def _sc_attend(table, idxm, wgtm):
    nq = idxm.shape[0] // 128
    rows_w = nq // _NW
    nch = rows_w // _CH
    mesh = plsc.VectorSubcoreMesh(core_axis_name="c", subcore_axis_name="s")

    @functools.partial(
        pl.kernel,
        out_type=jax.ShapeDtypeStruct((nq * 256,), jnp.float32),
        mesh=mesh,
        scratch_types=[
            pltpu.VMEM((_CH * 128,), jnp.int32),
            pltpu.VMEM((_CH * 512,), jnp.float32),
            pltpu.VMEM((_CH * 128, 128), jnp.float32),
            pltpu.VMEM((_CH * 256,), jnp.float32),
            pltpu.SemaphoreType.DMA,
        ],
    )
    def k(table_h, idx_h, wgt_h, out_h, idx_v, wgt_v, gath_v, out_v, sem):
        wid = lax.axis_index("s") * 2 + lax.axis_index("c")
        base = wid * rows_w

        def chunk(ci, carry):
            row0 = base + ci * _CH
            pltpu.sync_copy(idx_h.at[pl.ds(row0 * 128, _CH * 128)], idx_v)
            pltpu.sync_copy(wgt_h.at[pl.ds(row0 * 512, _CH * 512)], wgt_v)
            cps = []
            for r in range(_CH):
                cp = pltpu.make_async_copy(
                    table_h.at[idx_v.at[pl.ds(r * 128, 128)]],
                    gath_v.at[pl.ds(r * 128, 128)], sem)
                cp.start()
                cps.append(cp)
            for cp in cps:
                cp.wait()

            def qrow(r, c2):
                for m in range(_M):
                    gb = r * 128 + m * 16
                    wbase = r * 512 + m * 64
                    obase = r * 256 + m * 32
                    a0 = jnp.zeros((16,), jnp.float32)
                    a1 = jnp.zeros((16,), jnp.float32)
                    for j in range(16):
                        g = gb + j
                        if j % 4 == 0:
                            wv = wgt_v[pl.ds(wbase + (j // 4) * 16, 16)]
                        wj = (j % 4) * 4
                        w0 = _bcast(wv, wj)
                        w1 = _bcast(wv, wj + 1)
                        w2 = _bcast(wv, wj + 2)
                        a0 = (a0 + w0 * gath_v[g, pl.ds(0, 16)]
                              + w1 * gath_v[g, pl.ds(32, 16)]
                              + w2 * gath_v[g, pl.ds(64, 16)])
                        a1 = (a1 + w0 * gath_v[g, pl.ds(16, 16)]
                              + w1 * gath_v[g, pl.ds(48, 16)]
                              + w2 * gath_v[g, pl.ds(80, 16)])
                    out_v[pl.ds(obase, 16)] = a0
                    out_v[pl.ds(obase + 16, 16)] = a1
                return c2

            lax.fori_loop(0, _CH, qrow, 0)
            pltpu.sync_copy(out_v, out_h.at[pl.ds(row0 * 256, _CH * 256)])
            return carry

        lax.fori_loop(0, nch, chunk, 0)

    return k(table, idxm, wgtm)


def _make_table(value, n, lin):
    flat = value.reshape(n, lin, _M, _DH).transpose(0, 2, 1, 3)
    fp = flat.reshape(n * _M * lin // 2, 2 * _DH)
    nxt = jnp.concatenate([fp[1:], jnp.zeros((1, 2 * _DH), fp.dtype)], 0)
    return jnp.concatenate([fp, nxt], 1)


def _expand_ref(refpts, n, lq):
    r = refpts[:, :, :, 0]
    r = jnp.repeat(r, _P, axis=2)
    return jnp.tile(r, (1, 1, _M))


def _col_consts(shapes, lin):
    t = np.zeros((128,), np.float32)
    ib = np.zeros((128,), np.int32)
    starts = np.concatenate([[0], np.cumsum(shapes)[:-1]]).astype(np.int64)
    for c in range(128):
        m = c // 16
        l = (c // 4) % 4
        t[c] = shapes[l]
        ib[c] = m * lin + starts[l]
    return jnp.asarray(t).reshape(1, 128), jnp.asarray(ib).reshape(1, 128)


def kernel(video_src, audio_src, video_pos, audio_pos, video_reference_points,
           audio_reference_points, video_temporal_shapes, video_level_start_index,
           audio_temporal_shapes, audio_level_start_index, video_mask_flatten,
           audio_mask_flatten, params):
    pa = params['attn']
    n, lv, _ = video_src.shape
    la = audio_src.shape[1]

    vw_t = pa['value_w'].T
    vb = pa['value_b'].reshape(1, _D)
    cat_w = jnp.concatenate([pa['so_w'], pa['aw_w']], 0).T
    cat_b = jnp.concatenate([pa['so_b'], pa['aw_b']], 0).reshape(1, _D)
    ow_t = pa['out_w'].T
    ob = pa['out_b'].reshape(1, _D)
    g1 = params['norm1_g'].reshape(1, _D)
    b1 = params['norm1_b'].reshape(1, _D)
    w1_t = params['lin1_w'].T
    bb1 = params['lin1_b'].reshape(1, _DF)
    w2_t = params['lin2_w'].T
    bb2 = params['lin2_b'].reshape(1, _D)
    g2 = params['norm2_g'].reshape(1, _D)
    b2 = params['norm2_b'].reshape(1, _D)

    vmaskf = (1.0 - video_mask_flatten.astype(jnp.float32)).reshape(n, lv, 1)
    amaskf = (1.0 - audio_mask_flatten.astype(jnp.float32)).reshape(n, la, 1)
    vref_e = _expand_ref(video_reference_points, n, lv)
    aref_e = _expand_ref(audio_reference_points, n, la)
    vtcol, vibase = _col_consts(_VID, lv)
    atcol, aibase = _col_consts(_AUD, la)

    def attn(query, refe, val_src, maskf, tcol, ibase, lin):
        lq = query.shape[1]
        value = _value_proj(val_src, maskf, vw_t, vb)
        table = _make_table(value, n, lin)
        loc, aw, idxm, w0, w1, w2 = _samp(query, refe, cat_w, cat_b, tcol, ibase, lin)
        wgt = jnp.stack([w0, w1, w2, jnp.zeros_like(w0)], -1).reshape(n * lq * 512)
        acc = _sc_attend(table, idxm.reshape(n * lq * 128), wgt)
        return acc.reshape(n, lq, _D), loc, aw

    q1 = video_src + video_pos
    acc1, _, _ = attn(q1, vref_e, video_src, vmaskf, vtcol, vibase, lv)
    vs = _outln(acc1, video_src, ow_t, ob, g1, b1)

    q2 = audio_src + audio_pos
    acc2, _, _ = attn(q2, aref_e, audio_src, amaskf, atcol, aibase, la)
    au = _outln(acc2, audio_src, ow_t, ob, g1, b1)

    acc3, a_loc, a_w = attn(au, aref_e, vs, vmaskf, vtcol, vibase, lv)
    visual_attended_audio = _outffn(acc3, ow_t, ob, w1_t, bb1, w2_t, bb2, g2, b2)

    acc4, v_loc, v_w = attn(vs, vref_e, au, amaskf, atcol, aibase, la)
    audio_attended_visual = _outffn(acc4, ow_t, ob, w1_t, bb1, w2_t, bb2, g2, b2)

    v_loc = v_loc.reshape(n, lv, _M, _L, _P)
    v_w = v_w.reshape(n, lv, _M, _L, _P)
    a_loc = a_loc.reshape(n, la, _M, _L, _P)
    a_w = a_w.reshape(n, la, _M, _L, _P)
    return (audio_attended_visual, visual_attended_audio, v_loc, v_w, a_loc, a_w)


# speedup vs baseline: 3.0864x; 3.0864x over previous
import functools
import numpy as np
import jax
import jax.numpy as jnp
from jax import lax
from jax.experimental import pallas as pl
from jax.experimental.pallas import tpu as pltpu
from jax.experimental.pallas import tpu_sc as plsc

_D = 256
_M = 8
_L = 4
_P = 4
_DH = 32
_DF = 1024
_VID = (8192, 4096, 2048, 1024)
_AUD = (4096, 2048, 1024, 512)
_NW = 32
_CH = 4
_BQ = 512


def _value_body(src4_ref, mask4_ref, w4_ref, b4_ref, out_ref):
    x = src4_ref[0]
    v4 = jnp.dot(x, w4_ref[...], preferred_element_type=jnp.float32) + b4_ref[...]
    m4 = mask4_ref[0]
    me = jnp.broadcast_to(m4[:, None, :, None], (_BQ // 4, _M, 4, _DH))
    v4 = v4 * me.reshape(_BQ // 4, 4 * _D)
    for m in range(_M):
        out_ref[m, 0] = v4[:, m * 128:(m + 1) * 128]


def _value_proj(src4, mask4, w4, b4):
    n, lin4, _ = src4.shape
    return pl.pallas_call(
        _value_body,
        grid=(n, lin4 // (_BQ // 4)),
        in_specs=[
            pl.BlockSpec((1, _BQ // 4, 4 * _D), lambda i, j: (i, j, 0)),
            pl.BlockSpec((1, _BQ // 4, 4), lambda i, j: (i, j, 0)),
            pl.BlockSpec((4 * _D, 4 * _D), lambda i, j: (0, 0)),
            pl.BlockSpec((1, 4 * _D), lambda i, j: (0, 0)),
        ],
        out_specs=pl.BlockSpec((_M, 1, _BQ // 4, 128), lambda i, j: (0, i, j, 0)),
        out_shape=jax.ShapeDtypeStruct((_M, n, lin4, 128), jnp.float32),
    )(src4, mask4, w4, b4)


def _samp_body(q_ref, refe_ref, w_ref, b_ref, tcol_ref, ibase_ref,
               loc_ref, aw_ref, idx_ref, w0_ref, w1o_ref, w2_ref, *, lin):
    n = pl.program_id(0)
    q = q_ref[0]
    so_aw = jnp.dot(q, w_ref[...], preferred_element_type=jnp.float32) + b_ref[...]
    so = so_aw[:, :128]
    awl = so_aw[:, 128:]
    ri = lax.broadcasted_iota(jnp.int32, (128, 128), 0) // 16
    ci = lax.broadcasted_iota(jnp.int32, (128, 128), 1) // 16
    seg = (ri == ci).astype(jnp.float32)
    e = jnp.exp(awl)
    aw = e / jnp.dot(e, seg, preferred_element_type=jnp.float32)
    tcol = tcol_ref[...]
    loc = refe_ref[0] + so / tcol
    x = loc * tcol - 0.5
    x0f = jnp.floor(x)
    w1 = x - x0f
    t0 = (x0f >= 0.0) & (x0f <= tcol - 1.0)
    t1 = (x0f >= -1.0) & (x0f <= tcol - 2.0)
    wa = aw * jnp.where(t0, 1.0 - w1, jnp.where(t1, w1, 0.0))
    wb = aw * jnp.where(t0 & t1, w1, 0.0)
    r = jnp.clip(x0f, 0.0, tcol - 1.0).astype(jnp.int32)
    gidx = r + ibase_ref[...] + n * lin
    podd = (gidx & 1) == 1
    zero = jnp.zeros_like(wa)
    loc_ref[0] = loc.T
    aw_ref[0] = aw.T
    idx_ref[0] = gidx >> 1
    w0_ref[0] = jnp.where(podd, zero, wa)
    w1o_ref[0] = jnp.where(podd, wa, wb)
    w2_ref[0] = jnp.where(podd, wb, zero)


def _samp(query, refe, cat_w, cat_b, tcol, ibase, lin):
    n, lq, _ = query.shape
    grid = (n, lq // _BQ)
    blk = pl.BlockSpec((1, _BQ, 128), lambda i, j: (i, j, 0))
    blk_t = pl.BlockSpec((1, 128, _BQ), lambda i, j: (i, 0, j))
    out_shapes = [jax.ShapeDtypeStruct((n, 128, lq), jnp.float32)] * 2 + \
                 [jax.ShapeDtypeStruct((n, lq, 128), jnp.int32)] + \
                 [jax.ShapeDtypeStruct((n, lq, 128), jnp.float32)] * 3
    return pl.pallas_call(
        functools.partial(_samp_body, lin=lin),
        grid=grid,
        in_specs=[
            pl.BlockSpec((1, _BQ, _D), lambda i, j: (i, j, 0)),
            pl.BlockSpec((1, _BQ, 128), lambda i, j: (i, j, 0)),
            pl.BlockSpec((_D, _D), lambda i, j: (0, 0)),
            pl.BlockSpec((1, _D), lambda i, j: (0, 0)),
            pl.BlockSpec((1, 128), lambda i, j: (0, 0)),
            pl.BlockSpec((1, 128), lambda i, j: (0, 0)),
        ],
        out_specs=[blk_t, blk_t] + [blk] * 4,
        out_shape=out_shapes,
    )(query, refe, cat_w, cat_b, tcol, ibase)


def _outln_body(acc_ref, src_ref, w_ref, b_ref, g_ref, bb_ref, o_ref):
    a = acc_ref[0]
    y = jnp.dot(a, w_ref[...], preferred_element_type=jnp.float32) + b_ref[...]
    x = src_ref[0] + y
    mu = jnp.mean(x, -1, keepdims=True)
    var = jnp.mean((x - mu) ** 2, -1, keepdims=True)
    o_ref[0] = (x - mu) / jnp.sqrt(var + 1e-5) * g_ref[...] + bb_ref[...]


def _outln(acc, src, w_t, b, g, bb):
    n, lq, _ = acc.shape
    blk = pl.BlockSpec((1, _BQ, _D), lambda i, j: (i, j, 0))
    vec = pl.BlockSpec((1, _D), lambda i, j: (0, 0))
    return pl.pallas_call(
        _outln_body,
        grid=(n, lq // _BQ),
        in_specs=[blk, blk, pl.BlockSpec((_D, _D), lambda i, j: (0, 0)),
                  vec, vec, vec],
        out_specs=blk,
        out_shape=jax.ShapeDtypeStruct((n, lq, _D), jnp.float32),
    )(acc, src, w_t, b, g, bb)


def _outffn_body(acc_ref, ow_ref, ob_ref, w1_ref, b1_ref, w2_ref, b2_ref,
                 g_ref, bb_ref, o_ref):
    a = acc_ref[0]
    x = jnp.dot(a, ow_ref[...], preferred_element_type=jnp.float32) + ob_ref[...]
    h = jnp.maximum(
        jnp.dot(x, w1_ref[...], preferred_element_type=jnp.float32) + b1_ref[...],
        0.0)
    y = jnp.dot(h, w2_ref[...], preferred_element_type=jnp.float32) + b2_ref[...]
    x = x + y
    mu = jnp.mean(x, -1, keepdims=True)
    var = jnp.mean((x - mu) ** 2, -1, keepdims=True)
    o_ref[0] = (x - mu) / jnp.sqrt(var + 1e-5) * g_ref[...] + bb_ref[...]


def _outffn(acc, ow_t, ob, w1_t, b1, w2_t, b2, g, bb):
    n, lq, _ = acc.shape
    blk = pl.BlockSpec((1, _BQ, _D), lambda i, j: (i, j, 0))
    vec = pl.BlockSpec((1, _D), lambda i, j: (0, 0))
    return pl.pallas_call(
        _outffn_body,
        grid=(n, lq // _BQ),
        in_specs=[blk,
                  pl.BlockSpec((_D, _D), lambda i, j: (0, 0)), vec,
                  pl.BlockSpec((_D, _DF), lambda i, j: (0, 0)),
                  pl.BlockSpec((1, _DF), lambda i, j: (0, 0)),
                  pl.BlockSpec((_DF, _D), lambda i, j: (0, 0)), vec,
                  vec, vec],
        out_specs=blk,
        out_shape=jax.ShapeDtypeStruct((n, lq, _D), jnp.float32),
    )(acc, ow_t, ob, w1_t, b1, w2_t, b2, g, bb)


_GTR_DNUMS = lax.GatherDimensionNumbers(
    offset_dims=(), collapsed_slice_dims=(0,), start_index_map=(0,))


def _bcast(vec, j):
    idx = jnp.full((16, 1), j, jnp.int32)
    return lax.gather(vec, idx, _GTR_DNUMS, slice_sizes=(1,),
                      mode=lax.GatherScatterMode.PROMISE_IN_BOUNDS)


def _sc_attend(table, idxm, w0m, w1m, w2m):
    nq = idxm.shape[0]
    rows_w = nq // _NW
    nch = rows_w // _CH
    mesh = plsc.VectorSubcoreMesh(core_axis_name="c", subcore_axis_name="s")

    @functools.partial(
        pl.kernel,
        out_type=jax.ShapeDtypeStruct((nq * 2, 128), jnp.float32),
        mesh=mesh,
        scratch_types=[
            pltpu.VMEM((_CH, 128), jnp.int32),
            pltpu.VMEM((_CH, 128), jnp.float32),
            pltpu.VMEM((_CH, 128), jnp.float32),
            pltpu.VMEM((_CH, 128), jnp.float32),
            pltpu.VMEM((_CH * 128, 128), jnp.float32),
            pltpu.VMEM((_CH * 2, 128), jnp.float32),
            pltpu.SemaphoreType.DMA,
        ],
        compiler_params=pltpu.CompilerParams(use_tc_tiling_on_sc=True),
    )
    def k(table_h, idx_h, w0_h, w1_h, w2_h, out_h,
          idx_v, w0_v, w1_v, w2_v, gath_v, out_v, sem):
        wid = lax.axis_index("s") * 2 + lax.axis_index("c")
        base = wid * rows_w

        def chunk(ci, carry):
            row0 = base + ci * _CH
            pltpu.sync_copy(idx_h.at[pl.ds(row0, _CH)], idx_v)
            pltpu.sync_copy(w0_h.at[pl.ds(row0, _CH)], w0_v)
            pltpu.sync_copy(w1_h.at[pl.ds(row0, _CH)], w1_v)
            pltpu.sync_copy(w2_h.at[pl.ds(row0, _CH)], w2_v)
            cps = []
            for r in range(_CH):
                cp = pltpu.make_async_copy(
                    table_h.at[idx_v.at[r]],
                    gath_v.at[pl.ds(r * 128, 128)], sem)
                cp.start()
                cps.append(cp)
            for cp in cps:
                cp.wait()

            def qrow(r, c2):
                for m in range(_M):
                    gb = r * 128 + m * 16
                    orow = r * 2 + m // 4
                    ocb = (m % 4) * 32
                    wv0 = w0_v[r, pl.ds(m * 16, 16)]
                    wv1 = w1_v[r, pl.ds(m * 16, 16)]
                    wv2 = w2_v[r, pl.ds(m * 16, 16)]
                    a0 = jnp.zeros((16,), jnp.float32)
                    a1 = jnp.zeros((16,), jnp.float32)
                    for j in range(16):
                        g = gb + j
                        w0 = _bcast(wv0, j)
                        w1 = _bcast(wv1, j)
                        w2 = _bcast(wv2, j)
                        a0 = (a0 + w0 * gath_v[g, pl.ds(0, 16)]
                              + w1 * gath_v[g, pl.ds(32, 16)]
                              + w2 * gath_v[g, pl.ds(64, 16)])
                        a1 = (a1 + w0 * gath_v[g, pl.ds(16, 16)]
                              + w1 * gath_v[g, pl.ds(48, 16)]
                              + w2 * gath_v[g, pl.ds(80, 16)])
                    out_v[orow, pl.ds(ocb, 16)] = a0
                    out_v[orow, pl.ds(ocb + 16, 16)] = a1
                return c2

            lax.fori_loop(0, _CH, qrow, 0)
            pltpu.sync_copy(out_v, out_h.at[pl.ds(row0 * 2, _CH * 2)])
            return carry

        lax.fori_loop(0, nch, chunk, 0)

    return k(table, idxm, w0m, w1m, w2m)


def _make_table(fpa, n, lin):
    rq = _M * n * lin // 4
    a2 = fpa.reshape(rq, 128)
    right = jnp.concatenate([a2[1:, :64], jnp.zeros((1, 64), a2.dtype)], 0)
    ashift = jnp.concatenate([a2[:, 64:], right], 1)
    return jnp.stack([a2, ashift], 1).reshape(rq * 2, 128)


def _expand_ref(refpts, n, lq):
    r = refpts[:, :, :, 0]
    r = jnp.repeat(r, _P, axis=2)
    return jnp.tile(r, (1, 1, _M))


def _col_consts(shapes, lin, n):
    t = np.zeros((128,), np.float32)
    ib = np.zeros((128,), np.int32)
    starts = np.concatenate([[0], np.cumsum(shapes)[:-1]]).astype(np.int64)
    for c in range(128):
        m = c // 16
        l = (c // 4) % 4
        t[c] = shapes[l]
        ib[c] = m * n * lin + starts[l]
    return jnp.asarray(t).reshape(1, 128), jnp.asarray(ib).reshape(1, 128)


def kernel(video_src, audio_src, video_pos, audio_pos, video_reference_points,
           audio_reference_points, video_temporal_shapes, video_level_start_index,
           audio_temporal_shapes, audio_level_start_index, video_mask_flatten,
           audio_mask_flatten, params):
    pa = params['attn']
    n, lv, _ = video_src.shape
    la = audio_src.shape[1]

    vw_t = pa['value_w'].T
    wr = vw_t.reshape(1, _D, _M, 1, _DH)
    eye4 = jnp.eye(4, dtype=jnp.float32).reshape(4, 1, 1, 4, 1)
    w4 = (wr * eye4).reshape(4 * _D, 4 * _D)
    b4 = jnp.broadcast_to(
        pa['value_b'].reshape(_M, 1, _DH), (_M, 4, _DH)).reshape(1, 4 * _D)
    cat_w = jnp.concatenate([pa['so_w'], pa['aw_w']], 0).T
    cat_b = jnp.concatenate([pa['so_b'], pa['aw_b']], 0).reshape(1, _D)
    ow_t = pa['out_w'].T
    ob = pa['out_b'].reshape(1, _D)
    g1 = params['norm1_g'].reshape(1, _D)
    b1 = params['norm1_b'].reshape(1, _D)
    w1_t = params['lin1_w'].T
    bb1 = params['lin1_b'].reshape(1, _DF)
    w2_t = params['lin2_w'].T
    bb2 = params['lin2_b'].reshape(1, _D)
    g2 = params['norm2_g'].reshape(1, _D)
    b2 = params['norm2_b'].reshape(1, _D)

    vmaskf = (1.0 - video_mask_flatten.astype(jnp.float32))
    amaskf = (1.0 - audio_mask_flatten.astype(jnp.float32))
    vref_e = _expand_ref(video_reference_points, n, lv)
    aref_e = _expand_ref(audio_reference_points, n, la)
    vtcol, vibase = _col_consts(_VID, lv, n)
    atcol, aibase = _col_consts(_AUD, la, n)

    def attn(query, refe, val_src, maskf, tcol, ibase, lin):
        lq = query.shape[1]
        src4 = val_src.reshape(n, lin // 4, 4 * _D)
        mask4 = maskf.reshape(n, lin // 4, 4)
        fpa = _value_proj(src4, mask4, w4, b4)
        table = _make_table(fpa, n, lin)
        loc, aw, idxm, w0, w1, w2 = _samp(query, refe, cat_w, cat_b, tcol, ibase, lin)
        acc = _sc_attend(table, idxm.reshape(n * lq, 128),
                         w0.reshape(n * lq, 128), w1.reshape(n * lq, 128),
                         w2.reshape(n * lq, 128))
        return acc.reshape(n, lq, _D), loc, aw

    q1 = video_src + video_pos
    acc1, _, _ = attn(q1, vref_e, video_src, vmaskf, vtcol, vibase, lv)
    vs = _outln(acc1, video_src, ow_t, ob, g1, b1)

    q2 = audio_src + audio_pos
    acc2, _, _ = attn(q2, aref_e, audio_src, amaskf, atcol, aibase, la)
    au = _outln(acc2, audio_src, ow_t, ob, g1, b1)

    acc3, a_loc, a_w = attn(au, aref_e, vs, vmaskf, vtcol, vibase, lv)
    visual_attended_audio = _outffn(acc3, ow_t, ob, w1_t, bb1, w2_t, bb2, g2, b2)

    acc4, v_loc, v_w = attn(vs, vref_e, au, amaskf, atcol, aibase, la)
    audio_attended_visual = _outffn(acc4, ow_t, ob, w1_t, bb1, w2_t, bb2, g2, b2)

    def unt(x, lq):
        return x.reshape(n, _M, _L, _P, lq).transpose(0, 4, 1, 2, 3)

    return (audio_attended_visual, visual_attended_audio, unt(v_loc, lv),
            unt(v_w, lv), unt(a_loc, la), unt(a_w, la))


# speedup vs baseline: 5.4867x; 1.7777x over previous
import functools
import numpy as np
import jax
import jax.numpy as jnp
from jax import lax
from jax.experimental import pallas as pl
from jax.experimental.pallas import tpu as pltpu
from jax.experimental.pallas import tpu_sc as plsc

_D = 256
_M = 8
_L = 4
_P = 4
_DH = 32
_DF = 1024
_VID = (8192, 4096, 2048, 1024)
_AUD = (4096, 2048, 1024, 512)
_NW = 32
_CH = 4
_BQ = 512


def _value_body(src4_ref, mask4_ref, w4_ref, b4_ref, out_ref):
    x = src4_ref[0]
    v4 = jnp.dot(x, w4_ref[...], preferred_element_type=jnp.float32) + b4_ref[...]
    m4 = mask4_ref[0]
    me = jnp.broadcast_to(m4[:, None, :, None], (_BQ // 4, _M, 4, _DH))
    v4 = v4 * me.reshape(_BQ // 4, 4 * _D)
    for m in range(_M):
        out_ref[m, 0] = v4[:, m * 128:(m + 1) * 128]


def _value_proj(src4, mask4, w4, b4):
    n, lin4, _ = src4.shape
    return pl.pallas_call(
        _value_body,
        grid=(n, lin4 // (_BQ // 4)),
        in_specs=[
            pl.BlockSpec((1, _BQ // 4, 4 * _D), lambda i, j: (i, j, 0)),
            pl.BlockSpec((1, _BQ // 4, 4), lambda i, j: (i, j, 0)),
            pl.BlockSpec((4 * _D, 4 * _D), lambda i, j: (0, 0)),
            pl.BlockSpec((1, 4 * _D), lambda i, j: (0, 0)),
        ],
        out_specs=pl.BlockSpec((_M, 1, _BQ // 4, 128), lambda i, j: (0, i, j, 0)),
        out_shape=jax.ShapeDtypeStruct((_M, n, lin4, 128), jnp.float32),
    )(src4, mask4, w4, b4)


def _samp_body(q_ref, refe_ref, w_ref, b_ref, tcol_ref, ibase_ref,
               loc_ref, aw_ref, idx_ref, w0_ref, w1o_ref, w2_ref, *, lin):
    n = pl.program_id(0)
    q = q_ref[0]
    so_aw = jnp.dot(q, w_ref[...], preferred_element_type=jnp.float32) + b_ref[...]
    so = so_aw[:, :128]
    awl = so_aw[:, 128:]
    ri = lax.broadcasted_iota(jnp.int32, (128, 128), 0) // 16
    ci = lax.broadcasted_iota(jnp.int32, (128, 128), 1) // 16
    seg = (ri == ci).astype(jnp.float32)
    e = jnp.exp(awl)
    aw = e / jnp.dot(e, seg, preferred_element_type=jnp.float32)
    tcol = tcol_ref[...]
    loc = refe_ref[0] + so / tcol
    x = loc * tcol - 0.5
    x0f = jnp.floor(x)
    w1 = x - x0f
    t0 = (x0f >= 0.0) & (x0f <= tcol - 1.0)
    t1 = (x0f >= -1.0) & (x0f <= tcol - 2.0)
    wa = aw * jnp.where(t0, 1.0 - w1, jnp.where(t1, w1, 0.0))
    wb = aw * jnp.where(t0 & t1, w1, 0.0)
    r = jnp.clip(x0f, 0.0, tcol - 1.0).astype(jnp.int32)
    gidx = r + ibase_ref[...] + n * lin
    podd = (gidx & 1) == 1
    zero = jnp.zeros_like(wa)
    loc_ref[0] = loc.T
    aw_ref[0] = aw.T
    idx_ref[0] = gidx >> 1
    w0_ref[0] = jnp.where(podd, zero, wa)
    w1o_ref[0] = jnp.where(podd, wa, wb)
    w2_ref[0] = jnp.where(podd, wb, zero)


def _samp(query, refe, cat_w, cat_b, tcol, ibase, lin):
    n, lq, _ = query.shape
    grid = (n, lq // _BQ)
    blk = pl.BlockSpec((1, _BQ, 128), lambda i, j: (i, j, 0))
    blk_t = pl.BlockSpec((1, 128, _BQ), lambda i, j: (i, 0, j))
    out_shapes = [jax.ShapeDtypeStruct((n, 128, lq), jnp.float32)] * 2 + \
                 [jax.ShapeDtypeStruct((n, lq, 128), jnp.int32)] + \
                 [jax.ShapeDtypeStruct((n, lq, 128), jnp.float32)] * 3
    return pl.pallas_call(
        functools.partial(_samp_body, lin=lin),
        grid=grid,
        in_specs=[
            pl.BlockSpec((1, _BQ, _D), lambda i, j: (i, j, 0)),
            pl.BlockSpec((1, _BQ, 128), lambda i, j: (i, j, 0)),
            pl.BlockSpec((_D, _D), lambda i, j: (0, 0)),
            pl.BlockSpec((1, _D), lambda i, j: (0, 0)),
            pl.BlockSpec((1, 128), lambda i, j: (0, 0)),
            pl.BlockSpec((1, 128), lambda i, j: (0, 0)),
        ],
        out_specs=[blk_t, blk_t] + [blk] * 4,
        out_shape=out_shapes,
    )(query, refe, cat_w, cat_b, tcol, ibase)


def _outln_body(acc_ref, src_ref, w_ref, b_ref, g_ref, bb_ref, o_ref):
    a = acc_ref[0]
    y = jnp.dot(a, w_ref[...], preferred_element_type=jnp.float32) + b_ref[...]
    x = src_ref[0] + y
    mu = jnp.mean(x, -1, keepdims=True)
    var = jnp.mean((x - mu) ** 2, -1, keepdims=True)
    o_ref[0] = (x - mu) / jnp.sqrt(var + 1e-5) * g_ref[...] + bb_ref[...]


def _outln(acc, src, w_t, b, g, bb):
    n, lq, _ = acc.shape
    blk = pl.BlockSpec((1, _BQ, _D), lambda i, j: (i, j, 0))
    vec = pl.BlockSpec((1, _D), lambda i, j: (0, 0))
    return pl.pallas_call(
        _outln_body,
        grid=(n, lq // _BQ),
        in_specs=[blk, blk, pl.BlockSpec((_D, _D), lambda i, j: (0, 0)),
                  vec, vec, vec],
        out_specs=blk,
        out_shape=jax.ShapeDtypeStruct((n, lq, _D), jnp.float32),
    )(acc, src, w_t, b, g, bb)


def _outffn_body(acc_ref, ow_ref, ob_ref, w1_ref, b1_ref, w2_ref, b2_ref,
                 g_ref, bb_ref, o_ref):
    a = acc_ref[0]
    x = jnp.dot(a, ow_ref[...], preferred_element_type=jnp.float32) + ob_ref[...]
    h = jnp.maximum(
        jnp.dot(x, w1_ref[...], preferred_element_type=jnp.float32) + b1_ref[...],
        0.0)
    y = jnp.dot(h, w2_ref[...], preferred_element_type=jnp.float32) + b2_ref[...]
    x = x + y
    mu = jnp.mean(x, -1, keepdims=True)
    var = jnp.mean((x - mu) ** 2, -1, keepdims=True)
    o_ref[0] = (x - mu) / jnp.sqrt(var + 1e-5) * g_ref[...] + bb_ref[...]


def _outffn(acc, ow_t, ob, w1_t, b1, w2_t, b2, g, bb):
    n, lq, _ = acc.shape
    blk = pl.BlockSpec((1, _BQ, _D), lambda i, j: (i, j, 0))
    vec = pl.BlockSpec((1, _D), lambda i, j: (0, 0))
    return pl.pallas_call(
        _outffn_body,
        grid=(n, lq // _BQ),
        in_specs=[blk,
                  pl.BlockSpec((_D, _D), lambda i, j: (0, 0)), vec,
                  pl.BlockSpec((_D, _DF), lambda i, j: (0, 0)),
                  pl.BlockSpec((1, _DF), lambda i, j: (0, 0)),
                  pl.BlockSpec((_DF, _D), lambda i, j: (0, 0)), vec,
                  vec, vec],
        out_specs=blk,
        out_shape=jax.ShapeDtypeStruct((n, lq, _D), jnp.float32),
    )(acc, ow_t, ob, w1_t, b1, w2_t, b2, g, bb)


_GTR_DNUMS = lax.GatherDimensionNumbers(
    offset_dims=(), collapsed_slice_dims=(0,), start_index_map=(0,))


def _bcast(vec, j):
    idx = jnp.full((16, 1), j, jnp.int32)
    return lax.gather(vec, idx, _GTR_DNUMS, slice_sizes=(1,),
                      mode=lax.GatherScatterMode.PROMISE_IN_BOUNDS)


def _sc_attend(table, idxm, w0m, w1m, w2m):
    nq = idxm.shape[0]
    rows_w = nq // _NW
    ib = 8
    nblk = rows_w // ib
    mesh = plsc.VectorSubcoreMesh(core_axis_name="c", subcore_axis_name="s")

    @functools.partial(
        pl.kernel,
        out_type=jax.ShapeDtypeStruct((nq * 2, 128), jnp.float32),
        mesh=mesh,
        scratch_types=[
            pltpu.VMEM((2, ib, 128), jnp.int32),
            pltpu.VMEM((2, ib, 128), jnp.float32),
            pltpu.VMEM((2, ib, 128), jnp.float32),
            pltpu.VMEM((2, ib, 128), jnp.float32),
            pltpu.VMEM((4, 128, 128), jnp.float32),
            pltpu.VMEM((2, 2 * ib, 128), jnp.float32),
            pltpu.SemaphoreType.DMA,
            pltpu.SemaphoreType.DMA,
            pltpu.SemaphoreType.DMA,
            pltpu.SemaphoreType.DMA,
            pltpu.SemaphoreType.DMA,
            pltpu.SemaphoreType.DMA,
        ],
        compiler_params=pltpu.CompilerParams(use_tc_tiling_on_sc=True),
    )
    def k(table_h, idx_h, w0_h, w1_h, w2_h, out_h,
          idx_v, w0_v, w1_v, w2_v, gath_v, out_v,
          sem_iw, sem_g0, sem_g1, sem_g2, sem_g3, sem_o):
        wid = lax.axis_index("s") * 2 + lax.axis_index("c")
        base = wid * rows_w
        gsems = (sem_g0, sem_g1, sem_g2, sem_g3)

        def stage(blk, buf, do_async):
            row0 = base + blk * ib
            cps = [
                pltpu.make_async_copy(h.at[pl.ds(row0, ib)], v.at[buf], sem_iw)
                for h, v in ((idx_h, idx_v), (w0_h, w0_v),
                             (w1_h, w1_v), (w2_h, w2_v))
            ]
            for cp in cps:
                cp.start()
            if not do_async:
                for cp in cps:
                    cp.wait()

        def wait_stage(buf):
            for h, v in ((idx_h, idx_v), (w0_h, w0_v),
                         (w1_h, w1_v), (w2_h, w2_v)):
                pltpu.make_async_copy(h.at[pl.ds(0, ib)], v.at[buf],
                                      sem_iw).wait()

        def start_gather(buf, rsub, slot):
            pltpu.make_async_copy(
                table_h.at[idx_v.at[buf, rsub]],
                gath_v.at[slot], gsems[slot]).start()

        def wait_gather(slot):
            pltpu.make_async_copy(
                table_h.at[idx_v.at[0, 0]],
                gath_v.at[slot], gsems[slot]).wait()

        def out_write(buf, blk):
            pltpu.make_async_copy(
                out_v.at[buf],
                out_h.at[pl.ds((base + blk * ib) * 2, 2 * ib)], sem_o).start()

        def wait_out(buf):
            pltpu.make_async_copy(
                out_v.at[buf],
                out_h.at[pl.ds(0, 2 * ib)], sem_o).wait()

        stage(0, 0, False)
        if nblk > 1:
            stage(1, 1, True)
        start_gather(0, 0, 0)
        start_gather(0, 1, 1)

        def blk_body(cb, carry):
            pb = lax.rem(cb, 2)
            npb = 1 - pb

            @pl.when(cb >= 2)
            def _():
                wait_out(pb)

            for rsub in range(ib):
                slot = rsub % 4
                nslot = (rsub + 2) % 4
                r = cb * ib + rsub
                if rsub == ib - 2:
                    @pl.when(cb + 1 < nblk)
                    def _():
                        wait_stage(npb)
                        start_gather(npb, 0, nslot)
                elif rsub == ib - 1:
                    @pl.when(cb + 1 < nblk)
                    def _():
                        start_gather(npb, 1, nslot)
                else:
                    @pl.when(r + 2 < rows_w)
                    def _():
                        start_gather(pb, rsub + 2, nslot)

                wait_gather(slot)

                def head(m, c2):
                    orow = rsub * 2 + m // 4
                    ocb = (m % 4) * 32
                    wv0 = w0_v[pb, rsub, pl.ds(m * 16, 16)]
                    wv1 = w1_v[pb, rsub, pl.ds(m * 16, 16)]
                    wv2 = w2_v[pb, rsub, pl.ds(m * 16, 16)]
                    a0 = jnp.zeros((16,), jnp.float32)
                    a1 = jnp.zeros((16,), jnp.float32)
                    for j in range(16):
                        g = m * 16 + j
                        w0 = _bcast(wv0, j)
                        w1 = _bcast(wv1, j)
                        w2 = _bcast(wv2, j)
                        a0 = (a0 + w0 * gath_v[slot, g, pl.ds(0, 16)]
                              + w1 * gath_v[slot, g, pl.ds(32, 16)]
                              + w2 * gath_v[slot, g, pl.ds(64, 16)])
                        a1 = (a1 + w0 * gath_v[slot, g, pl.ds(16, 16)]
                              + w1 * gath_v[slot, g, pl.ds(48, 16)]
                              + w2 * gath_v[slot, g, pl.ds(80, 16)])
                    out_v[pb, orow, pl.ds(ocb, 16)] = a0
                    out_v[pb, orow, pl.ds(ocb + 16, 16)] = a1
                    return c2

                lax.fori_loop(0, _M, head, 0)

            out_write(pb, cb)

            @pl.when(cb + 2 < nblk)
            def _():
                stage(cb + 2, pb, True)

            return carry

        lax.fori_loop(0, nblk, blk_body, 0)
        wait_out(0)
        if nblk > 1:
            wait_out(1)

    return k(table, idxm, w0m, w1m, w2m)


def _make_table(fpa, n, lin):
    rq = _M * n * lin // 4
    a2 = fpa.reshape(rq, 128)
    right = jnp.concatenate([a2[1:, :64], jnp.zeros((1, 64), a2.dtype)], 0)
    ashift = jnp.concatenate([a2[:, 64:], right], 1)
    return jnp.stack([a2, ashift], 1).reshape(rq * 2, 128)


def _expand_ref(refpts, n, lq):
    r = refpts[:, :, :, 0]
    r = jnp.repeat(r, _P, axis=2)
    return jnp.tile(r, (1, 1, _M))


def _col_consts(shapes, lin, n):
    t = np.zeros((128,), np.float32)
    ib = np.zeros((128,), np.int32)
    starts = np.concatenate([[0], np.cumsum(shapes)[:-1]]).astype(np.int64)
    for c in range(128):
        m = c // 16
        l = (c // 4) % 4
        t[c] = shapes[l]
        ib[c] = m * n * lin + starts[l]
    return jnp.asarray(t).reshape(1, 128), jnp.asarray(ib).reshape(1, 128)


def kernel(video_src, audio_src, video_pos, audio_pos, video_reference_points,
           audio_reference_points, video_temporal_shapes, video_level_start_index,
           audio_temporal_shapes, audio_level_start_index, video_mask_flatten,
           audio_mask_flatten, params):
    pa = params['attn']
    n, lv, _ = video_src.shape
    la = audio_src.shape[1]

    vw_t = pa['value_w'].T
    wr = vw_t.reshape(1, _D, _M, 1, _DH)
    eye4 = jnp.eye(4, dtype=jnp.float32).reshape(4, 1, 1, 4, 1)
    w4 = (wr * eye4).reshape(4 * _D, 4 * _D)
    b4 = jnp.broadcast_to(
        pa['value_b'].reshape(_M, 1, _DH), (_M, 4, _DH)).reshape(1, 4 * _D)
    cat_w = jnp.concatenate([pa['so_w'], pa['aw_w']], 0).T
    cat_b = jnp.concatenate([pa['so_b'], pa['aw_b']], 0).reshape(1, _D)
    ow_t = pa['out_w'].T
    ob = pa['out_b'].reshape(1, _D)
    g1 = params['norm1_g'].reshape(1, _D)
    b1 = params['norm1_b'].reshape(1, _D)
    w1_t = params['lin1_w'].T
    bb1 = params['lin1_b'].reshape(1, _DF)
    w2_t = params['lin2_w'].T
    bb2 = params['lin2_b'].reshape(1, _D)
    g2 = params['norm2_g'].reshape(1, _D)
    b2 = params['norm2_b'].reshape(1, _D)

    vmaskf = (1.0 - video_mask_flatten.astype(jnp.float32))
    amaskf = (1.0 - audio_mask_flatten.astype(jnp.float32))
    vref_e = _expand_ref(video_reference_points, n, lv)
    aref_e = _expand_ref(audio_reference_points, n, la)
    vtcol, vibase = _col_consts(_VID, lv, n)
    atcol, aibase = _col_consts(_AUD, la, n)

    def attn(query, refe, val_src, maskf, tcol, ibase, lin):
        lq = query.shape[1]
        src4 = val_src.reshape(n, lin // 4, 4 * _D)
        mask4 = maskf.reshape(n, lin // 4, 4)
        fpa = _value_proj(src4, mask4, w4, b4)
        table = _make_table(fpa, n, lin)
        loc, aw, idxm, w0, w1, w2 = _samp(query, refe, cat_w, cat_b, tcol, ibase, lin)
        acc = _sc_attend(table, idxm.reshape(n * lq, 128),
                         w0.reshape(n * lq, 128), w1.reshape(n * lq, 128),
                         w2.reshape(n * lq, 128))
        return acc.reshape(n, lq, _D), loc, aw

    q1 = video_src + video_pos
    acc1, _, _ = attn(q1, vref_e, video_src, vmaskf, vtcol, vibase, lv)
    vs = _outln(acc1, video_src, ow_t, ob, g1, b1)

    q2 = audio_src + audio_pos
    acc2, _, _ = attn(q2, aref_e, audio_src, amaskf, atcol, aibase, la)
    au = _outln(acc2, audio_src, ow_t, ob, g1, b1)

    acc3, a_loc, a_w = attn(au, aref_e, vs, vmaskf, vtcol, vibase, lv)
    visual_attended_audio = _outffn(acc3, ow_t, ob, w1_t, bb1, w2_t, bb2, g2, b2)

    acc4, v_loc, v_w = attn(vs, vref_e, au, amaskf, atcol, aibase, la)
    audio_attended_visual = _outffn(acc4, ow_t, ob, w1_t, bb1, w2_t, bb2, g2, b2)

    def unt(x, lq):
        return x.reshape(n, _M, _L, _P, lq).transpose(0, 4, 1, 2, 3)

    return (audio_attended_visual, visual_attended_audio, unt(v_loc, lv),
            unt(v_w, lv), unt(a_loc, la), unt(a_w, la))


# speedup vs baseline: 5.7338x; 1.0450x over previous
import functools
import numpy as np
import jax
import jax.numpy as jnp
from jax import lax
from jax.experimental import pallas as pl
from jax.experimental.pallas import tpu as pltpu
from jax.experimental.pallas import tpu_sc as plsc

_D = 256
_M = 8
_L = 4
_P = 4
_DH = 32
_DF = 1024
_VID = (8192, 4096, 2048, 1024)
_AUD = (4096, 2048, 1024, 512)
_NW = 32
_CH = 4
_BQ = 512


def _value_body(src4_ref, mask4_ref, w4_ref, b4_ref, out_ref):
    x = src4_ref[0]
    v4 = jnp.dot(x, w4_ref[...], preferred_element_type=jnp.float32) + b4_ref[...]
    m4 = mask4_ref[0]
    me = jnp.broadcast_to(m4[:, None, :, None], (_BQ // 4, _M, 4, _DH))
    v4 = v4 * me.reshape(_BQ // 4, 4 * _D)
    for m in range(_M):
        out_ref[m, 0] = v4[:, m * 128:(m + 1) * 128]


def _value_proj(src4, mask4, w4, b4):
    n, lin4, _ = src4.shape
    return pl.pallas_call(
        _value_body,
        grid=(n, lin4 // (_BQ // 4)),
        in_specs=[
            pl.BlockSpec((1, _BQ // 4, 4 * _D), lambda i, j: (i, j, 0)),
            pl.BlockSpec((1, _BQ // 4, 4), lambda i, j: (i, j, 0)),
            pl.BlockSpec((4 * _D, 4 * _D), lambda i, j: (0, 0)),
            pl.BlockSpec((1, 4 * _D), lambda i, j: (0, 0)),
        ],
        out_specs=pl.BlockSpec((_M, 1, _BQ // 4, 128), lambda i, j: (0, i, j, 0)),
        out_shape=jax.ShapeDtypeStruct((_M, n, lin4, 128), jnp.float32),
    )(src4, mask4, w4, b4)


def _samp_body(q_ref, refe_ref, w_ref, b_ref, tcol_ref, ibase_ref,
               loc_ref, aw_ref, idx_ref, w0_ref, w1o_ref, w2_ref, *, lin):
    n = pl.program_id(0)
    q = q_ref[0]
    so_aw = jnp.dot(q, w_ref[...], preferred_element_type=jnp.float32) + b_ref[...]
    so = so_aw[:, :128]
    awl = so_aw[:, 128:]
    ri = lax.broadcasted_iota(jnp.int32, (128, 128), 0) // 16
    ci = lax.broadcasted_iota(jnp.int32, (128, 128), 1) // 16
    seg = (ri == ci).astype(jnp.float32)
    e = jnp.exp(awl)
    aw = e / jnp.dot(e, seg, preferred_element_type=jnp.float32)
    tcol = tcol_ref[...]
    loc = refe_ref[0] + so / tcol
    x = loc * tcol - 0.5
    x0f = jnp.floor(x)
    w1 = x - x0f
    t0 = (x0f >= 0.0) & (x0f <= tcol - 1.0)
    t1 = (x0f >= -1.0) & (x0f <= tcol - 2.0)
    wa = aw * jnp.where(t0, 1.0 - w1, jnp.where(t1, w1, 0.0))
    wb = aw * jnp.where(t0 & t1, w1, 0.0)
    r = jnp.clip(x0f, 0.0, tcol - 1.0).astype(jnp.int32)
    gidx = r + ibase_ref[...] + n * lin
    podd = (gidx & 1) == 1
    zero = jnp.zeros_like(wa)
    loc_ref[0] = loc.T
    aw_ref[0] = aw.T
    idx_ref[0] = gidx >> 1
    w0_ref[0] = jnp.where(podd, zero, wa)
    w1o_ref[0] = jnp.where(podd, wa, wb)
    w2_ref[0] = jnp.where(podd, wb, zero)


def _samp(query, refe, cat_w, cat_b, tcol, ibase, lin):
    n, lq, _ = query.shape
    grid = (n, lq // _BQ)
    blk = pl.BlockSpec((1, _BQ, 128), lambda i, j: (i, j, 0))
    blk_t = pl.BlockSpec((1, 128, _BQ), lambda i, j: (i, 0, j))
    out_shapes = [jax.ShapeDtypeStruct((n, 128, lq), jnp.float32)] * 2 + \
                 [jax.ShapeDtypeStruct((n, lq, 128), jnp.int32)] + \
                 [jax.ShapeDtypeStruct((n, lq, 128), jnp.float32)] * 3
    return pl.pallas_call(
        functools.partial(_samp_body, lin=lin),
        grid=grid,
        in_specs=[
            pl.BlockSpec((1, _BQ, _D), lambda i, j: (i, j, 0)),
            pl.BlockSpec((1, _BQ, 128), lambda i, j: (i, j, 0)),
            pl.BlockSpec((_D, _D), lambda i, j: (0, 0)),
            pl.BlockSpec((1, _D), lambda i, j: (0, 0)),
            pl.BlockSpec((1, 128), lambda i, j: (0, 0)),
            pl.BlockSpec((1, 128), lambda i, j: (0, 0)),
        ],
        out_specs=[blk_t, blk_t] + [blk] * 4,
        out_shape=out_shapes,
    )(query, refe, cat_w, cat_b, tcol, ibase)


def _outln_body(acc_ref, src_ref, w_ref, b_ref, g_ref, bb_ref, o_ref):
    a = acc_ref[0]
    y = jnp.dot(a, w_ref[...], preferred_element_type=jnp.float32) + b_ref[...]
    x = src_ref[0] + y
    mu = jnp.mean(x, -1, keepdims=True)
    var = jnp.mean((x - mu) ** 2, -1, keepdims=True)
    o_ref[0] = (x - mu) / jnp.sqrt(var + 1e-5) * g_ref[...] + bb_ref[...]


def _outln(acc, src, w_t, b, g, bb):
    n, lq, _ = acc.shape
    blk = pl.BlockSpec((1, _BQ, _D), lambda i, j: (i, j, 0))
    vec = pl.BlockSpec((1, _D), lambda i, j: (0, 0))
    return pl.pallas_call(
        _outln_body,
        grid=(n, lq // _BQ),
        in_specs=[blk, blk, pl.BlockSpec((_D, _D), lambda i, j: (0, 0)),
                  vec, vec, vec],
        out_specs=blk,
        out_shape=jax.ShapeDtypeStruct((n, lq, _D), jnp.float32),
    )(acc, src, w_t, b, g, bb)


def _outffn_body(acc_ref, ow_ref, ob_ref, w1_ref, b1_ref, w2_ref, b2_ref,
                 g_ref, bb_ref, o_ref):
    a = acc_ref[0]
    x = jnp.dot(a, ow_ref[...], preferred_element_type=jnp.float32) + ob_ref[...]
    h = jnp.maximum(
        jnp.dot(x, w1_ref[...], preferred_element_type=jnp.float32) + b1_ref[...],
        0.0)
    y = jnp.dot(h, w2_ref[...], preferred_element_type=jnp.float32) + b2_ref[...]
    x = x + y
    mu = jnp.mean(x, -1, keepdims=True)
    var = jnp.mean((x - mu) ** 2, -1, keepdims=True)
    o_ref[0] = (x - mu) / jnp.sqrt(var + 1e-5) * g_ref[...] + bb_ref[...]


def _outffn(acc, ow_t, ob, w1_t, b1, w2_t, b2, g, bb):
    n, lq, _ = acc.shape
    blk = pl.BlockSpec((1, _BQ, _D), lambda i, j: (i, j, 0))
    vec = pl.BlockSpec((1, _D), lambda i, j: (0, 0))
    return pl.pallas_call(
        _outffn_body,
        grid=(n, lq // _BQ),
        in_specs=[blk,
                  pl.BlockSpec((_D, _D), lambda i, j: (0, 0)), vec,
                  pl.BlockSpec((_D, _DF), lambda i, j: (0, 0)),
                  pl.BlockSpec((1, _DF), lambda i, j: (0, 0)),
                  pl.BlockSpec((_DF, _D), lambda i, j: (0, 0)), vec,
                  vec, vec],
        out_specs=blk,
        out_shape=jax.ShapeDtypeStruct((n, lq, _D), jnp.float32),
    )(acc, ow_t, ob, w1_t, b1, w2_t, b2, g, bb)


_GTR_DNUMS = lax.GatherDimensionNumbers(
    offset_dims=(), collapsed_slice_dims=(0,), start_index_map=(0,))


def _bcast(vec, j):
    idx = jnp.full((16, 1), j, jnp.int32)
    return lax.gather(vec, idx, _GTR_DNUMS, slice_sizes=(1,),
                      mode=lax.GatherScatterMode.PROMISE_IN_BOUNDS)


def _sc_attend(table, idxm, w0m, w1m, w2m):
    nq = idxm.shape[0]
    rows_w = nq // _NW
    ib = 8
    nblk = rows_w // ib
    mesh = plsc.VectorSubcoreMesh(core_axis_name="c", subcore_axis_name="s")

    @functools.partial(
        pl.kernel,
        out_type=jax.ShapeDtypeStruct((nq * 2, 128), jnp.float32),
        mesh=mesh,
        scratch_types=[
            pltpu.VMEM((2, ib, 128), jnp.int32),
            pltpu.VMEM((2, ib, 128), jnp.float32),
            pltpu.VMEM((2, ib, 128), jnp.float32),
            pltpu.VMEM((2, ib, 128), jnp.float32),
            pltpu.VMEM((4, 128, 128), jnp.float32),
            pltpu.VMEM((2, 2 * ib, 128), jnp.float32),
            pltpu.SemaphoreType.DMA,
            pltpu.SemaphoreType.DMA,
            pltpu.SemaphoreType.DMA,
            pltpu.SemaphoreType.DMA,
            pltpu.SemaphoreType.DMA,
            pltpu.SemaphoreType.DMA,
        ],
        compiler_params=pltpu.CompilerParams(use_tc_tiling_on_sc=True),
    )
    def k(table_h, idx_h, w0_h, w1_h, w2_h, out_h,
          idx_v, w0_v, w1_v, w2_v, gath_v, out_v,
          sem_iw, sem_g0, sem_g1, sem_g2, sem_g3, sem_o):
        wid = lax.axis_index("s") * 2 + lax.axis_index("c")
        base = wid * rows_w
        gsems = (sem_g0, sem_g1, sem_g2, sem_g3)

        def stage(blk, buf, do_async):
            row0 = base + blk * ib
            cps = [
                pltpu.make_async_copy(h.at[pl.ds(row0, ib)], v.at[buf], sem_iw)
                for h, v in ((idx_h, idx_v), (w0_h, w0_v),
                             (w1_h, w1_v), (w2_h, w2_v))
            ]
            for cp in cps:
                cp.start()
            if not do_async:
                for cp in cps:
                    cp.wait()

        def wait_stage(buf):
            for h, v in ((idx_h, idx_v), (w0_h, w0_v),
                         (w1_h, w1_v), (w2_h, w2_v)):
                pltpu.make_async_copy(h.at[pl.ds(0, ib)], v.at[buf],
                                      sem_iw).wait()

        def start_gather(buf, rsub, slot):
            pltpu.make_async_copy(
                table_h.at[idx_v.at[buf, rsub]],
                gath_v.at[slot], gsems[slot]).start()

        def wait_gather(slot):
            pltpu.make_async_copy(
                table_h.at[idx_v.at[0, 0]],
                gath_v.at[slot], gsems[slot]).wait()

        def out_write(buf, blk):
            pltpu.make_async_copy(
                out_v.at[buf],
                out_h.at[pl.ds((base + blk * ib) * 2, 2 * ib)], sem_o).start()

        def wait_out(buf):
            pltpu.make_async_copy(
                out_v.at[buf],
                out_h.at[pl.ds(0, 2 * ib)], sem_o).wait()

        stage(0, 0, False)
        if nblk > 1:
            stage(1, 1, True)
        start_gather(0, 0, 0)
        start_gather(0, 1, 1)
        start_gather(0, 2, 2)

        def blk_body(cb, carry):
            pb = lax.rem(cb, 2)
            npb = 1 - pb

            @pl.when(cb >= 2)
            def _():
                wait_out(pb)

            for rsub in range(ib):
                slot = rsub % 4
                nslot = (rsub + 3) % 4
                r = cb * ib + rsub
                if rsub == ib - 3:
                    @pl.when(cb + 1 < nblk)
                    def _():
                        wait_stage(npb)
                        start_gather(npb, 0, nslot)
                elif rsub >= ib - 2:
                    @pl.when(cb + 1 < nblk)
                    def _():
                        start_gather(npb, rsub - (ib - 3), nslot)
                else:
                    @pl.when(r + 3 < rows_w)
                    def _():
                        start_gather(pb, rsub + 3, nslot)

                wait_gather(slot)

                def head(m, c2):
                    orow = rsub * 2 + m // 4
                    ocb = (m % 4) * 32
                    wv0 = w0_v[pb, rsub, pl.ds(m * 16, 16)]
                    wv1 = w1_v[pb, rsub, pl.ds(m * 16, 16)]
                    wv2 = w2_v[pb, rsub, pl.ds(m * 16, 16)]
                    a0 = jnp.zeros((16,), jnp.float32)
                    a1 = jnp.zeros((16,), jnp.float32)
                    for j in range(16):
                        g = m * 16 + j
                        w0 = _bcast(wv0, j)
                        w1 = _bcast(wv1, j)
                        w2 = _bcast(wv2, j)
                        a0 = (a0 + w0 * gath_v[slot, g, pl.ds(0, 16)]
                              + w1 * gath_v[slot, g, pl.ds(32, 16)]
                              + w2 * gath_v[slot, g, pl.ds(64, 16)])
                        a1 = (a1 + w0 * gath_v[slot, g, pl.ds(16, 16)]
                              + w1 * gath_v[slot, g, pl.ds(48, 16)]
                              + w2 * gath_v[slot, g, pl.ds(80, 16)])
                    out_v[pb, orow, pl.ds(ocb, 16)] = a0
                    out_v[pb, orow, pl.ds(ocb + 16, 16)] = a1
                    return c2

                lax.fori_loop(0, _M, head, 0)

            out_write(pb, cb)

            @pl.when(cb + 2 < nblk)
            def _():
                stage(cb + 2, pb, True)

            return carry

        lax.fori_loop(0, nblk, blk_body, 0)
        wait_out(0)
        if nblk > 1:
            wait_out(1)

    return k(table, idxm, w0m, w1m, w2m)


def _make_table(fpa, n, lin):
    rq = _M * n * lin // 4
    a2 = fpa.reshape(rq, 128)
    right = jnp.concatenate([a2[1:, :64], jnp.zeros((1, 64), a2.dtype)], 0)
    ashift = jnp.concatenate([a2[:, 64:], right], 1)
    return jnp.stack([a2, ashift], 1).reshape(rq * 2, 128)


def _expand_ref(refpts, n, lq):
    r = refpts[:, :, :, 0]
    r = jnp.repeat(r, _P, axis=2)
    return jnp.tile(r, (1, 1, _M))


def _col_consts(shapes, lin, n):
    t = np.zeros((128,), np.float32)
    ib = np.zeros((128,), np.int32)
    starts = np.concatenate([[0], np.cumsum(shapes)[:-1]]).astype(np.int64)
    for c in range(128):
        m = c // 16
        l = (c // 4) % 4
        t[c] = shapes[l]
        ib[c] = m * n * lin + starts[l]
    return jnp.asarray(t).reshape(1, 128), jnp.asarray(ib).reshape(1, 128)


def kernel(video_src, audio_src, video_pos, audio_pos, video_reference_points,
           audio_reference_points, video_temporal_shapes, video_level_start_index,
           audio_temporal_shapes, audio_level_start_index, video_mask_flatten,
           audio_mask_flatten, params):
    pa = params['attn']
    n, lv, _ = video_src.shape
    la = audio_src.shape[1]

    vw_t = pa['value_w'].T
    wr = vw_t.reshape(1, _D, _M, 1, _DH)
    eye4 = jnp.eye(4, dtype=jnp.float32).reshape(4, 1, 1, 4, 1)
    w4 = (wr * eye4).reshape(4 * _D, 4 * _D)
    b4 = jnp.broadcast_to(
        pa['value_b'].reshape(_M, 1, _DH), (_M, 4, _DH)).reshape(1, 4 * _D)
    cat_w = jnp.concatenate([pa['so_w'], pa['aw_w']], 0).T
    cat_b = jnp.concatenate([pa['so_b'], pa['aw_b']], 0).reshape(1, _D)
    ow_t = pa['out_w'].T
    ob = pa['out_b'].reshape(1, _D)
    g1 = params['norm1_g'].reshape(1, _D)
    b1 = params['norm1_b'].reshape(1, _D)
    w1_t = params['lin1_w'].T
    bb1 = params['lin1_b'].reshape(1, _DF)
    w2_t = params['lin2_w'].T
    bb2 = params['lin2_b'].reshape(1, _D)
    g2 = params['norm2_g'].reshape(1, _D)
    b2 = params['norm2_b'].reshape(1, _D)

    vmaskf = (1.0 - video_mask_flatten.astype(jnp.float32))
    amaskf = (1.0 - audio_mask_flatten.astype(jnp.float32))
    vref_e = _expand_ref(video_reference_points, n, lv)
    aref_e = _expand_ref(audio_reference_points, n, la)
    vtcol, vibase = _col_consts(_VID, lv, n)
    atcol, aibase = _col_consts(_AUD, la, n)

    def attn(query, refe, val_src, maskf, tcol, ibase, lin):
        lq = query.shape[1]
        src4 = val_src.reshape(n, lin // 4, 4 * _D)
        mask4 = maskf.reshape(n, lin // 4, 4)
        fpa = _value_proj(src4, mask4, w4, b4)
        table = _make_table(fpa, n, lin)
        loc, aw, idxm, w0, w1, w2 = _samp(query, refe, cat_w, cat_b, tcol, ibase, lin)
        acc = _sc_attend(table, idxm.reshape(n * lq, 128),
                         w0.reshape(n * lq, 128), w1.reshape(n * lq, 128),
                         w2.reshape(n * lq, 128))
        return acc.reshape(n, lq, _D), loc, aw

    q1 = video_src + video_pos
    acc1, _, _ = attn(q1, vref_e, video_src, vmaskf, vtcol, vibase, lv)
    vs = _outln(acc1, video_src, ow_t, ob, g1, b1)

    q2 = audio_src + audio_pos
    acc2, _, _ = attn(q2, aref_e, audio_src, amaskf, atcol, aibase, la)
    au = _outln(acc2, audio_src, ow_t, ob, g1, b1)

    acc3, a_loc, a_w = attn(au, aref_e, vs, vmaskf, vtcol, vibase, lv)
    visual_attended_audio = _outffn(acc3, ow_t, ob, w1_t, bb1, w2_t, bb2, g2, b2)

    acc4, v_loc, v_w = attn(vs, vref_e, au, amaskf, atcol, aibase, la)
    audio_attended_visual = _outffn(acc4, ow_t, ob, w1_t, bb1, w2_t, bb2, g2, b2)

    def unt(x, lq):
        return x.reshape(n, _M, _L, _P, lq).transpose(0, 4, 1, 2, 3)

    return (audio_attended_visual, visual_attended_audio, unt(v_loc, lv),
            unt(v_w, lv), unt(a_loc, la), unt(a_w, la))


# speedup vs baseline: 5.7384x; 1.0008x over previous
import functools
import numpy as np
import jax
import jax.numpy as jnp
from jax import lax
from jax.experimental import pallas as pl
from jax.experimental.pallas import tpu as pltpu
from jax.experimental.pallas import tpu_sc as plsc

_D = 256
_M = 8
_L = 4
_P = 4
_DH = 32
_DF = 1024
_VID = (8192, 4096, 2048, 1024)
_AUD = (4096, 2048, 1024, 512)
_NW = 32
_BQ = 512


def _value_body(src4_ref, mask4_ref, w4_ref, b4_ref, out_ref):
    x = src4_ref[0]
    v4 = jnp.dot(x, w4_ref[...], preferred_element_type=jnp.float32) + b4_ref[...]
    m4 = mask4_ref[0]
    me = jnp.broadcast_to(m4[:, None, :, None], (_BQ // 4, _M, 4, _DH))
    v4 = v4 * me.reshape(_BQ // 4, 4 * _D)
    for m in range(_M):
        out_ref[m, 0] = v4[:, m * 128:(m + 1) * 128]


def _value_proj(src4, mask4, w4, b4):
    n, lin4, _ = src4.shape
    return pl.pallas_call(
        _value_body,
        grid=(n, lin4 // (_BQ // 4)),
        in_specs=[
            pl.BlockSpec((1, _BQ // 4, 4 * _D), lambda i, j: (i, j, 0)),
            pl.BlockSpec((1, _BQ // 4, 4), lambda i, j: (i, j, 0)),
            pl.BlockSpec((4 * _D, 4 * _D), lambda i, j: (0, 0)),
            pl.BlockSpec((1, 4 * _D), lambda i, j: (0, 0)),
        ],
        out_specs=pl.BlockSpec((_M, 1, _BQ // 4, 128), lambda i, j: (0, i, j, 0)),
        out_shape=jax.ShapeDtypeStruct((_M, n, lin4, 128), jnp.float32),
    )(src4, mask4, w4, b4)


def _samp_body(q_ref, refe_ref, w_ref, b_ref, tcol_ref, ibase_ref,
               loc_ref, aw_ref, idx_ref, w0_ref, w1o_ref, w2_ref, *, lin):
    n = pl.program_id(0)
    q = q_ref[0]
    so_aw = jnp.dot(q, w_ref[...], preferred_element_type=jnp.float32) + b_ref[...]
    so = so_aw[:, :128]
    awl = so_aw[:, 128:]
    ri = lax.broadcasted_iota(jnp.int32, (128, 128), 0) // 16
    ci = lax.broadcasted_iota(jnp.int32, (128, 128), 1) // 16
    seg = (ri == ci).astype(jnp.float32)
    e = jnp.exp(awl)
    aw = e / jnp.dot(e, seg, preferred_element_type=jnp.float32)
    tcol = tcol_ref[...]
    loc = refe_ref[0] + so / tcol
    x = loc * tcol - 0.5
    x0f = jnp.floor(x)
    w1 = x - x0f
    t0 = (x0f >= 0.0) & (x0f <= tcol - 1.0)
    t1 = (x0f >= -1.0) & (x0f <= tcol - 2.0)
    wa = aw * jnp.where(t0, 1.0 - w1, jnp.where(t1, w1, 0.0))
    wb = aw * jnp.where(t0 & t1, w1, 0.0)
    r = jnp.clip(x0f, 0.0, tcol - 1.0).astype(jnp.int32)
    gidx = r + ibase_ref[...] + n * lin
    podd = (gidx & 1) == 1
    zero = jnp.zeros_like(wa)
    loc_ref[0] = loc.T
    aw_ref[0] = aw.T
    idx_ref[0] = gidx >> 1
    w0_ref[0] = jnp.where(podd, zero, wa)
    w1o_ref[0] = jnp.where(podd, wa, wb)
    w2_ref[0] = jnp.where(podd, wb, zero)


def _samp(query, refe, cat_w, cat_b, tcol, ibase, lin):
    n, lq, _ = query.shape
    grid = (n, lq // _BQ)
    blk = pl.BlockSpec((1, _BQ, 128), lambda i, j: (i, j, 0))
    blk_t = pl.BlockSpec((1, 128, _BQ), lambda i, j: (i, 0, j))
    out_shapes = [jax.ShapeDtypeStruct((n, 128, lq), jnp.float32)] * 2 + \
                 [jax.ShapeDtypeStruct((n, lq, 128), jnp.int32)] + \
                 [jax.ShapeDtypeStruct((n, lq, 128), jnp.float32)] * 3
    return pl.pallas_call(
        functools.partial(_samp_body, lin=lin),
        grid=grid,
        in_specs=[
            pl.BlockSpec((1, _BQ, _D), lambda i, j: (i, j, 0)),
            pl.BlockSpec((1, _BQ, 128), lambda i, j: (i, j, 0)),
            pl.BlockSpec((_D, _D), lambda i, j: (0, 0)),
            pl.BlockSpec((1, _D), lambda i, j: (0, 0)),
            pl.BlockSpec((1, 128), lambda i, j: (0, 0)),
            pl.BlockSpec((1, 128), lambda i, j: (0, 0)),
        ],
        out_specs=[blk_t, blk_t] + [blk] * 4,
        out_shape=out_shapes,
    )(query, refe, cat_w, cat_b, tcol, ibase)


def _outln_body(acc_ref, src_ref, w_ref, b_ref, g_ref, bb_ref, o_ref):
    a = acc_ref[0]
    y = jnp.dot(a, w_ref[...], preferred_element_type=jnp.float32) + b_ref[...]
    x = src_ref[0] + y
    mu = jnp.mean(x, -1, keepdims=True)
    var = jnp.mean((x - mu) ** 2, -1, keepdims=True)
    o_ref[0] = (x - mu) / jnp.sqrt(var + 1e-5) * g_ref[...] + bb_ref[...]


def _outln(acc, src, w_t, b, g, bb):
    n, lq, _ = acc.shape
    blk = pl.BlockSpec((1, _BQ, _D), lambda i, j: (i, j, 0))
    vec = pl.BlockSpec((1, _D), lambda i, j: (0, 0))
    return pl.pallas_call(
        _outln_body,
        grid=(n, lq // _BQ),
        in_specs=[blk, blk, pl.BlockSpec((_D, _D), lambda i, j: (0, 0)),
                  vec, vec, vec],
        out_specs=blk,
        out_shape=jax.ShapeDtypeStruct((n, lq, _D), jnp.float32),
    )(acc, src, w_t, b, g, bb)


def _outffn_body(acc_ref, ow_ref, ob_ref, w1_ref, b1_ref, w2_ref, b2_ref,
                 g_ref, bb_ref, o_ref):
    a = acc_ref[0]
    x = jnp.dot(a, ow_ref[...], preferred_element_type=jnp.float32) + ob_ref[...]
    h = jnp.maximum(
        jnp.dot(x, w1_ref[...], preferred_element_type=jnp.float32) + b1_ref[...],
        0.0)
    y = jnp.dot(h, w2_ref[...], preferred_element_type=jnp.float32) + b2_ref[...]
    x = x + y
    mu = jnp.mean(x, -1, keepdims=True)
    var = jnp.mean((x - mu) ** 2, -1, keepdims=True)
    o_ref[0] = (x - mu) / jnp.sqrt(var + 1e-5) * g_ref[...] + bb_ref[...]


def _outffn(acc, ow_t, ob, w1_t, b1, w2_t, b2, g, bb):
    n, lq, _ = acc.shape
    blk = pl.BlockSpec((1, _BQ, _D), lambda i, j: (i, j, 0))
    vec = pl.BlockSpec((1, _D), lambda i, j: (0, 0))
    return pl.pallas_call(
        _outffn_body,
        grid=(n, lq // _BQ),
        in_specs=[blk,
                  pl.BlockSpec((_D, _D), lambda i, j: (0, 0)), vec,
                  pl.BlockSpec((_D, _DF), lambda i, j: (0, 0)),
                  pl.BlockSpec((1, _DF), lambda i, j: (0, 0)),
                  pl.BlockSpec((_DF, _D), lambda i, j: (0, 0)), vec,
                  vec, vec],
        out_specs=blk,
        out_shape=jax.ShapeDtypeStruct((n, lq, _D), jnp.float32),
    )(acc, ow_t, ob, w1_t, b1, w2_t, b2, g, bb)


_GTR_DNUMS = lax.GatherDimensionNumbers(
    offset_dims=(), collapsed_slice_dims=(0,), start_index_map=(0,))


def _bcast(vec, j):
    idx = jnp.full((16, 1), j, jnp.int32)
    return lax.gather(vec, idx, _GTR_DNUMS, slice_sizes=(1,),
                      mode=lax.GatherScatterMode.PROMISE_IN_BOUNDS)


def _sc_attend(table, idxm, w0m, w1m, w2m):
    nq = idxm.shape[0]
    rows_w = nq // _NW
    ib = 8
    nblk = rows_w // ib
    mesh = plsc.VectorSubcoreMesh(core_axis_name="c", subcore_axis_name="s")

    @functools.partial(
        pl.kernel,
        out_type=jax.ShapeDtypeStruct((nq * 2, 128), jnp.float32),
        mesh=mesh,
        scratch_types=[
            pltpu.VMEM((2, ib, 128), jnp.int32),
            pltpu.VMEM((2, ib, 128), jnp.float32),
            pltpu.VMEM((2, ib, 128), jnp.float32),
            pltpu.VMEM((2, ib, 128), jnp.float32),
            pltpu.VMEM((4, 128, 128), jnp.float32),
            pltpu.VMEM((2, 2 * ib, 128), jnp.float32),
            pltpu.SemaphoreType.DMA,
            pltpu.SemaphoreType.DMA,
            pltpu.SemaphoreType.DMA,
            pltpu.SemaphoreType.DMA,
            pltpu.SemaphoreType.DMA,
            pltpu.SemaphoreType.DMA,
        ],
        compiler_params=pltpu.CompilerParams(use_tc_tiling_on_sc=True),
    )
    def k(table_h, idx_h, w0_h, w1_h, w2_h, out_h,
          idx_v, w0_v, w1_v, w2_v, gath_v, out_v,
          sem_iw, sem_g0, sem_g1, sem_g2, sem_g3, sem_o):
        wid = lax.axis_index("s") * 2 + lax.axis_index("c")
        base = wid * rows_w
        gsems = (sem_g0, sem_g1, sem_g2, sem_g3)

        def stage(blk, buf, do_async):
            row0 = base + blk * ib
            cps = [
                pltpu.make_async_copy(h.at[pl.ds(row0, ib)], v.at[buf], sem_iw)
                for h, v in ((idx_h, idx_v), (w0_h, w0_v),
                             (w1_h, w1_v), (w2_h, w2_v))
            ]
            for cp in cps:
                cp.start()
            if not do_async:
                for cp in cps:
                    cp.wait()

        def wait_stage(buf):
            for h, v in ((idx_h, idx_v), (w0_h, w0_v),
                         (w1_h, w1_v), (w2_h, w2_v)):
                pltpu.make_async_copy(h.at[pl.ds(0, ib)], v.at[buf],
                                      sem_iw).wait()

        def start_gather(buf, rsub, slot):
            pltpu.make_async_copy(
                table_h.at[idx_v.at[buf, rsub]],
                gath_v.at[slot], gsems[slot]).start()

        def wait_gather(slot):
            pltpu.make_async_copy(
                table_h.at[idx_v.at[0, 0]],
                gath_v.at[slot], gsems[slot]).wait()

        def out_write(buf, blk):
            pltpu.make_async_copy(
                out_v.at[buf],
                out_h.at[pl.ds((base + blk * ib) * 2, 2 * ib)], sem_o).start()

        def wait_out(buf):
            pltpu.make_async_copy(
                out_v.at[buf],
                out_h.at[pl.ds(0, 2 * ib)], sem_o).wait()

        stage(0, 0, False)
        if nblk > 1:
            stage(1, 1, True)
        start_gather(0, 0, 0)
        start_gather(0, 1, 1)
        start_gather(0, 2, 2)

        def blk_body(cb, carry):
            pb = lax.rem(cb, 2)
            npb = 1 - pb

            @pl.when(cb >= 2)
            def _():
                wait_out(pb)

            for rsub in range(ib):
                slot = rsub % 4
                nslot = (rsub + 3) % 4
                r = cb * ib + rsub
                if rsub == ib - 3:
                    @pl.when(cb + 1 < nblk)
                    def _():
                        wait_stage(npb)
                        start_gather(npb, 0, nslot)
                elif rsub >= ib - 2:
                    @pl.when(cb + 1 < nblk)
                    def _():
                        start_gather(npb, rsub - (ib - 3), nslot)
                else:
                    @pl.when(r + 3 < rows_w)
                    def _():
                        start_gather(pb, rsub + 3, nslot)

                wait_gather(slot)

                def head(m, c2):
                    orow = rsub * 2 + m // 4
                    ocb = (m % 4) * 32
                    wv0 = w0_v[pb, rsub, pl.ds(m * 16, 16)]
                    wv1 = w1_v[pb, rsub, pl.ds(m * 16, 16)]
                    wv2 = w2_v[pb, rsub, pl.ds(m * 16, 16)]
                    a0 = jnp.zeros((16,), jnp.float32)
                    a1 = jnp.zeros((16,), jnp.float32)
                    for j in range(16):
                        g = m * 16 + j
                        w0 = _bcast(wv0, j)
                        w1 = _bcast(wv1, j)
                        w2 = _bcast(wv2, j)
                        a0 = (a0 + w0 * gath_v[slot, g, pl.ds(0, 16)]
                              + w1 * gath_v[slot, g, pl.ds(32, 16)]
                              + w2 * gath_v[slot, g, pl.ds(64, 16)])
                        a1 = (a1 + w0 * gath_v[slot, g, pl.ds(16, 16)]
                              + w1 * gath_v[slot, g, pl.ds(48, 16)]
                              + w2 * gath_v[slot, g, pl.ds(80, 16)])
                    out_v[pb, orow, pl.ds(ocb, 16)] = a0
                    out_v[pb, orow, pl.ds(ocb + 16, 16)] = a1
                    return c2

                lax.fori_loop(0, _M, head, 0)

            out_write(pb, cb)

            @pl.when(cb + 2 < nblk)
            def _():
                stage(cb + 2, pb, True)

            return carry

        lax.fori_loop(0, nblk, blk_body, 0)
        wait_out(0)
        if nblk > 1:
            wait_out(1)

    return k(table, idxm, w0m, w1m, w2m)


def _make_table(fpa, n, lin):
    rq = _M * n * lin // 4
    a2 = fpa.reshape(rq, 128)
    right = jnp.concatenate([a2[1:, :64], jnp.zeros((1, 64), a2.dtype)], 0)
    ashift = jnp.concatenate([a2[:, 64:], right], 1)
    return jnp.stack([a2, ashift], 1).reshape(rq * 2, 128)


def _expand_ref(refpts, n, lq):
    r = refpts[:, :, :, 0]
    r = jnp.repeat(r, _P, axis=2)
    return jnp.tile(r, (1, 1, _M))


def _col_consts(shapes, lin, n):
    t = np.zeros((128,), np.float32)
    ib = np.zeros((128,), np.int32)
    starts = np.concatenate([[0], np.cumsum(shapes)[:-1]]).astype(np.int64)
    for c in range(128):
        m = c // 16
        l = (c // 4) % 4
        t[c] = shapes[l]
        ib[c] = m * n * lin + starts[l]
    return jnp.asarray(t).reshape(1, 128), jnp.asarray(ib).reshape(1, 128)


def kernel(video_src, audio_src, video_pos, audio_pos, video_reference_points,
           audio_reference_points, video_temporal_shapes, video_level_start_index,
           audio_temporal_shapes, audio_level_start_index, video_mask_flatten,
           audio_mask_flatten, params):
    pa = params['attn']
    n, lv, _ = video_src.shape
    la = audio_src.shape[1]

    vw_t = pa['value_w'].T
    wr = vw_t.reshape(1, _D, _M, 1, _DH)
    eye4 = jnp.eye(4, dtype=jnp.float32).reshape(4, 1, 1, 4, 1)
    w4 = (wr * eye4).reshape(4 * _D, 4 * _D)
    b4 = jnp.broadcast_to(
        pa['value_b'].reshape(_M, 1, _DH), (_M, 4, _DH)).reshape(1, 4 * _D)
    cat_w = jnp.concatenate([pa['so_w'], pa['aw_w']], 0).T
    cat_b = jnp.concatenate([pa['so_b'], pa['aw_b']], 0).reshape(1, _D)
    ow_t = pa['out_w'].T
    ob = pa['out_b'].reshape(1, _D)
    g1 = params['norm1_g'].reshape(1, _D)
    b1 = params['norm1_b'].reshape(1, _D)
    w1_t = params['lin1_w'].T
    bb1 = params['lin1_b'].reshape(1, _DF)
    w2_t = params['lin2_w'].T
    bb2 = params['lin2_b'].reshape(1, _D)
    g2 = params['norm2_g'].reshape(1, _D)
    b2 = params['norm2_b'].reshape(1, _D)

    vmaskf = (1.0 - video_mask_flatten.astype(jnp.float32))
    amaskf = (1.0 - audio_mask_flatten.astype(jnp.float32))
    vref_e = _expand_ref(video_reference_points, n, lv)
    aref_e = _expand_ref(audio_reference_points, n, la)
    vtcol, vibase = _col_consts(_VID, lv, n)
    atcol, aibase = _col_consts(_AUD, la, n)

    def attn(query, refe, val_src, maskf, tcol, ibase, lin):
        lq = query.shape[1]
        src4 = val_src.reshape(n, lin // 4, 4 * _D)
        mask4 = maskf.reshape(n, lin // 4, 4)
        fpa = _value_proj(src4, mask4, w4, b4)
        table = _make_table(fpa, n, lin)
        loc, aw, idxm, w0, w1, w2 = _samp(query, refe, cat_w, cat_b, tcol, ibase, lin)
        acc = _sc_attend(table, idxm.reshape(n * lq, 128),
                         w0.reshape(n * lq, 128), w1.reshape(n * lq, 128),
                         w2.reshape(n * lq, 128))
        return acc.reshape(n, lq, _D), loc, aw

    q1 = video_src + video_pos
    acc1, _, _ = attn(q1, vref_e, video_src, vmaskf, vtcol, vibase, lv)
    vs = _outln(acc1, video_src, ow_t, ob, g1, b1)

    q2 = audio_src + audio_pos
    acc2, _, _ = attn(q2, aref_e, audio_src, amaskf, atcol, aibase, la)
    au = _outln(acc2, audio_src, ow_t, ob, g1, b1)

    acc3, a_loc, a_w = attn(au, aref_e, vs, vmaskf, vtcol, vibase, lv)
    visual_attended_audio = _outffn(acc3, ow_t, ob, w1_t, bb1, w2_t, bb2, g2, b2)

    acc4, v_loc, v_w = attn(vs, vref_e, au, amaskf, atcol, aibase, la)
    audio_attended_visual = _outffn(acc4, ow_t, ob, w1_t, bb1, w2_t, bb2, g2, b2)

    def unt(x, lq):
        return x.reshape(n, _M, _L, _P, lq).transpose(0, 4, 1, 2, 3)

    return (audio_attended_visual, visual_attended_audio, unt(v_loc, lv),
            unt(v_w, lv), unt(a_loc, la), unt(a_w, la))
